# baseline (speedup 1.0000x reference)
import functools

import jax
import jax.numpy as jnp
from jax import lax
from jax.experimental import pallas as pl
from jax.experimental.pallas import tpu as pltpu
from jax.experimental.pallas import tpu_sc as plsc

VOCAB = 100000
DIM = 64
BATCH = 16384
SCALE = 10.0
EPS = 1e-8


def _sc_gather3(iword, oword, onword, center_table, context_table):
    info = plsc.get_sparse_core_info()
    nc, ns = info.num_cores, info.num_subcores
    nw = nc * ns
    b_per_w = BATCH // nw

    row_t = jax.ShapeDtypeStruct((BATCH, DIM), jnp.float32)

    @functools.partial(
        pl.kernel,
        out_type=[row_t, row_t, row_t],
        mesh=plsc.VectorSubcoreMesh(core_axis_name="c", subcore_axis_name="s"),
        compiler_params=pltpu.CompilerParams(use_tc_tiling_on_sc=False),
        scratch_types=[
            pltpu.VMEM((b_per_w,), jnp.int32),
            pltpu.VMEM((b_per_w, DIM), jnp.float32),
            pltpu.SemaphoreType.DMA,
        ],
    )
    def gather_k(iw_hbm, ow_hbm, onw_hbm, ctr_hbm, ctx_hbm,
                 iv_hbm, ov_hbm, onv_hbm, idx_v, rows_v, sem):
        wid = lax.axis_index("s") * nc + lax.axis_index("c")
        base = wid * b_per_w
        for idx_hbm, tab_hbm, out_hbm in (
            (iw_hbm, ctr_hbm, iv_hbm),
            (ow_hbm, ctx_hbm, ov_hbm),
            (onw_hbm, ctx_hbm, onv_hbm),
        ):
            pltpu.sync_copy(idx_hbm.at[pl.ds(base, b_per_w)], idx_v)
            pltpu.async_copy(tab_hbm.at[idx_v], rows_v, sem).wait()
            pltpu.sync_copy(rows_v, out_hbm.at[pl.ds(base, b_per_w)])

    return gather_k(iword, oword, onword, center_table, context_table)


def _loss_body(iv_ref, ov_ref, onv_ref, out_ref):
    @pl.when(pl.program_id(0) == 0)
    def _init():
        out_ref[...] = jnp.zeros_like(out_ref)

    iv = iv_ref[...]
    ov = ov_ref[...]
    onv = onv_ref[...]
    ni = jnp.sqrt(jnp.sum(iv * iv, axis=1)) + EPS
    no = jnp.sqrt(jnp.sum(ov * ov, axis=1)) + EPS
    non = jnp.sqrt(jnp.sum(onv * onv, axis=1)) + EPS
    dio = jnp.sum(iv * ov, axis=1)
    dion = jnp.sum(iv * onv, axis=1)
    dist_io = 1.0 - dio / (ni * no)
    dist_ion = 1.0 - dion / (ni * non)
    x1 = -SCALE * dist_io
    x2 = SCALE * dist_ion
    oloss = jnp.minimum(x1, 0.0) - jnp.log1p(jnp.exp(-jnp.abs(x1)))
    nloss = jnp.minimum(x2, 0.0) - jnp.log1p(jnp.exp(-jnp.abs(x2)))
    total = -jnp.sum(oloss + nloss) / BATCH
    out_ref[...] += jnp.broadcast_to(total, (1, 1))


def kernel(iword, oword, onword, center_table, context_table):
    iword = iword.astype(jnp.int32)
    oword = oword.astype(jnp.int32)
    onword = onword.astype(jnp.int32)
    iv, ov, onv = _sc_gather3(iword, oword, onword, center_table, context_table)
    blk = 1024
    out = pl.pallas_call(
        _loss_body,
        grid=(BATCH // blk,),
        in_specs=[pl.BlockSpec((blk, DIM), lambda i: (i, 0))] * 3,
        out_specs=pl.BlockSpec((1, 1), lambda i: (0, 0)),
        out_shape=jax.ShapeDtypeStruct((1, 1), jnp.float32),
    )(iv, ov, onv)
    return out[0, 0]


# speedup vs baseline: 1.1051x; 1.1051x over previous
import functools

import jax
import jax.numpy as jnp
from jax import lax
from jax.experimental import pallas as pl
from jax.experimental.pallas import tpu as pltpu
from jax.experimental.pallas import tpu_sc as plsc

VOCAB = 100000
DIM = 64
BATCH = 16384
SCALE = 10.0
EPS = 1e-8


def _sc_gather3(iword, oword, onword, center_table, context_table):
    info = plsc.get_sparse_core_info()
    nc, ns = info.num_cores, info.num_subcores
    nw = nc * ns
    b_per_w = BATCH // nw

    row_t = jax.ShapeDtypeStruct((BATCH, DIM), jnp.float32)

    @functools.partial(
        pl.kernel,
        out_type=[row_t, row_t, row_t],
        mesh=plsc.VectorSubcoreMesh(core_axis_name="c", subcore_axis_name="s"),
        compiler_params=pltpu.CompilerParams(use_tc_tiling_on_sc=False),
        scratch_types=[
            pltpu.VMEM((b_per_w,), jnp.int32),
            pltpu.VMEM((b_per_w, DIM), jnp.float32),
            pltpu.SemaphoreType.DMA,
        ],
    )
    def gather_k(iw_hbm, ow_hbm, onw_hbm, ctr_hbm, ctx_hbm,
                 iv_hbm, ov_hbm, onv_hbm, idx_v, rows_v, sem):
        wid = lax.axis_index("s") * nc + lax.axis_index("c")
        base = wid * b_per_w
        for idx_hbm, tab_hbm, out_hbm in (
            (iw_hbm, ctr_hbm, iv_hbm),
            (ow_hbm, ctx_hbm, ov_hbm),
            (onw_hbm, ctx_hbm, onv_hbm),
        ):
            pltpu.sync_copy(idx_hbm.at[pl.ds(base, b_per_w)], idx_v)
            pltpu.async_copy(tab_hbm.at[idx_v], rows_v, sem).wait()
            pltpu.sync_copy(rows_v, out_hbm.at[pl.ds(base, b_per_w)])

    return gather_k(iword, oword, onword, center_table, context_table)


def _hsum(x):
    return jnp.sum(x[:, :DIM], axis=1), jnp.sum(x[:, DIM:], axis=1)


def _loss_body(iv_ref, ov_ref, onv_ref, out_ref):
    @pl.when(pl.program_id(0) == 0)
    def _init():
        out_ref[...] = jnp.zeros_like(out_ref)

    iv = iv_ref[...]
    ov = ov_ref[...]
    onv = onv_ref[...]

    def per_half(nia, dioa, diona, noa, nona):
        ni = jnp.sqrt(nia) + EPS
        no = jnp.sqrt(noa) + EPS
        non = jnp.sqrt(nona) + EPS
        x1 = -SCALE * (1.0 - dioa / (ni * no))
        x2 = SCALE * (1.0 - diona / (ni * non))
        oloss = jnp.minimum(x1, 0.0) - jnp.log1p(jnp.exp(-jnp.abs(x1)))
        nloss = jnp.minimum(x2, 0.0) - jnp.log1p(jnp.exp(-jnp.abs(x2)))
        return jnp.sum(oloss + nloss)

    nia, nib = _hsum(iv * iv)
    noa, nob = _hsum(ov * ov)
    nona, nonb = _hsum(onv * onv)
    dioa, diob = _hsum(iv * ov)
    diona, dionb = _hsum(iv * onv)
    total = per_half(nia, dioa, diona, noa, nona) + per_half(
        nib, diob, dionb, nob, nonb)
    out_ref[...] += jnp.broadcast_to(-total / BATCH, (1, 1))


def kernel(iword, oword, onword, center_table, context_table):
    iword = iword.astype(jnp.int32)
    oword = oword.astype(jnp.int32)
    onword = onword.astype(jnp.int32)
    iv, ov, onv = _sc_gather3(iword, oword, onword, center_table, context_table)
    rows = BATCH // 2
    iv, ov, onv = (x.reshape(rows, 2 * DIM) for x in (iv, ov, onv))
    blk = 1024
    out = pl.pallas_call(
        _loss_body,
        grid=(rows // blk,),
        in_specs=[pl.BlockSpec((blk, 2 * DIM), lambda i: (i, 0))] * 3,
        out_specs=pl.BlockSpec((1, 1), lambda i: (0, 0)),
        out_shape=jax.ShapeDtypeStruct((1, 1), jnp.float32),
    )(iv, ov, onv)
    return out[0, 0]
